# baseline (speedup 1.0000x reference)
import functools

import jax
import jax.numpy as jnp
from jax import lax
from jax.experimental import pallas as pl
from jax.experimental.pallas import tpu as pltpu
from jax.experimental.pallas import tpu_sc as plsc

NC, NS = 2, 16
NW = NC * NS
LANES = 16
CHUNK = 80
NB = 4
NO = 2


def _embed_call(ids_flat, token_table2, pos_table, n, s, e):
    per_w = n // NW
    n_chunks = per_w // CHUNK
    assert n_chunks % NB == 0, (n_chunks, NB)
    n_groups = n_chunks // NB

    mesh = plsc.VectorSubcoreMesh(
        core_axis_name="c", subcore_axis_name="s", num_cores=NC, num_subcores=NS
    )

    @functools.partial(
        pl.kernel,
        out_type=jax.ShapeDtypeStruct((n, e), jnp.float32),
        mesh=mesh,
        scratch_types=[
            pltpu.VMEM((per_w,), jnp.int32),
            pltpu.VMEM((s, e), jnp.float32),
        ]
        + [pltpu.VMEM((CHUNK,), jnp.int32) for _ in range(NB)]
        + [pltpu.VMEM((CHUNK, 2 * e), jnp.float32) for _ in range(NB)]
        + [pltpu.VMEM((CHUNK, e), jnp.float32) for _ in range(NO)]
        + [pltpu.SemaphoreType.DMA for _ in range(NB + NO)],
        compiler_params=pltpu.CompilerParams(needs_layout_passes=False),
    )
    def embed(ids_hbm, tok_hbm, pos_hbm, out_hbm, idx_v, pos_v, *bufs):
        half = bufs[:NB]
        rows = bufs[NB : 2 * NB]
        outs = bufs[2 * NB : 2 * NB + NO]
        gsem = bufs[2 * NB + NO : 3 * NB + NO]
        wsem = bufs[3 * NB + NO : 3 * NB + 2 * NO]
        wid = lax.axis_index("s") * NC + lax.axis_index("c")
        base_w = wid * per_w
        pltpu.sync_copy(pos_hbm, pos_v)
        pltpu.sync_copy(ids_hbm.at[pl.ds(base_w, per_w)], idx_v)
        iota16 = lax.iota(jnp.int32, LANES)

        def fire_gather(k, b):
            o = k * CHUNK
            for v in range(CHUNK // LANES):
                sl = pl.ds(v * LANES, LANES)
                half[b][sl] = lax.shift_right_logical(idx_v[pl.ds(o + v * LANES, LANES)], 1)
            pltpu.async_copy(tok_hbm.at[half[b]], rows[b], gsem[b])

        def drain_gather(b):
            pltpu.make_async_copy(tok_hbm.at[pl.ds(0, CHUNK)], rows[b], gsem[b]).wait()

        def drain_writeback(ob):
            pltpu.make_async_copy(outs[ob], out_hbm.at[pl.ds(0, CHUNK)], wsem[ob]).wait()

        for b in range(NB - 1):
            fire_gather(b, b)

        def group_body(g, carry):
            for b in range(NB):
                k = g * NB + b
                drain_gather(b)
                ob = b % NO

                @pl.when(k >= NO)
                def _():
                    drain_writeback(ob)

                def block_body(j, carry2):
                    o = j * LANES
                    ids16 = idx_v[pl.ds(k * CHUNK + o, LANES)]
                    colbase = (ids16 & 1) * e
                    rvec = o + iota16
                    svec = lax.rem(k * CHUNK + o + iota16, s)
                    for c in range(e):
                        cvec = colbase + c
                        vals = plsc.load_gather(rows[b], [rvec, cvec])
                        pvals = plsc.load_gather(pos_v, [svec, jnp.full((LANES,), c, jnp.int32)])
                        plsc.store_scatter(
                            outs[ob], [rvec, jnp.full((LANES,), c, jnp.int32)], vals + pvals
                        )
                    return carry2

                lax.fori_loop(0, CHUNK // LANES, block_body, 0)
                pltpu.async_copy(
                    outs[ob], out_hbm.at[pl.ds(base_w + k * CHUNK, CHUNK)], wsem[ob]
                )
                kn = k + NB - 1
                bp = (b + NB - 1) % NB

                @pl.when(kn < n_chunks)
                def _():
                    fire_gather(kn, bp)

            return carry

        lax.fori_loop(0, n_groups, group_body, 0)
        for ob in range(NO):
            drain_writeback(ob)

    return embed(ids_flat, token_table2, pos_table)


def kernel(token_ids, token_table, pos_table):
    b, s = token_ids.shape
    v, e = token_table.shape
    n = b * s
    ids_flat = token_ids.reshape(n).astype(jnp.int32)
    tok2 = token_table.reshape(v // 2, 2 * e)
    out = _embed_call(ids_flat, tok2, pos_table, n, s, e)
    return out.reshape(b, s, e)


# speedup vs baseline: 2.7495x; 2.7495x over previous
import functools

import jax
import jax.numpy as jnp
from jax import lax
from jax.experimental import pallas as pl
from jax.experimental.pallas import tpu as pltpu
from jax.experimental.pallas import tpu_sc as plsc

NC, NS = 2, 16
NW = NC * NS
LANES = 16
CHUNK = 80
NB = 4
NO = 2


def _embed_call(ids_flat, token_table2, pos_table, n, s, e):
    per_w = n // NW
    n_chunks = per_w // CHUNK
    assert n_chunks % NB == 0, (n_chunks, NB)
    n_groups = n_chunks // NB

    mesh = plsc.VectorSubcoreMesh(
        core_axis_name="c", subcore_axis_name="s", num_cores=NC, num_subcores=NS
    )

    @functools.partial(
        pl.kernel,
        out_type=jax.ShapeDtypeStruct((n, e), jnp.float32),
        mesh=mesh,
        scratch_types=[
            pltpu.VMEM((per_w,), jnp.int32),
            pltpu.VMEM((s, e), jnp.float32),
        ]
        + [pltpu.VMEM((CHUNK,), jnp.int32) for _ in range(NB)]
        + [pltpu.VMEM((CHUNK, 2 * e), jnp.float32) for _ in range(NB)]
        + [pltpu.VMEM((CHUNK, e), jnp.float32) for _ in range(NO)]
        + [pltpu.SemaphoreType.DMA for _ in range(NB + NO)],
        compiler_params=pltpu.CompilerParams(needs_layout_passes=False),
    )
    def embed(ids_hbm, tok_hbm, pos_hbm, out_hbm, idx_v, pos_v, *bufs):
        half = bufs[:NB]
        rows = bufs[NB : 2 * NB]
        outs = bufs[2 * NB : 2 * NB + NO]
        gsem = bufs[2 * NB + NO : 3 * NB + NO]
        wsem = bufs[3 * NB + NO : 3 * NB + 2 * NO]
        wid = lax.axis_index("s") * NC + lax.axis_index("c")
        base_w = wid * per_w
        pltpu.sync_copy(pos_hbm, pos_v)
        pltpu.sync_copy(ids_hbm.at[pl.ds(base_w, per_w)], idx_v)
        iota16 = lax.iota(jnp.int32, LANES)

        def fire_gather(k, b):
            o = k * CHUNK
            for v in range(CHUNK // LANES):
                sl = pl.ds(v * LANES, LANES)
                half[b][sl] = lax.shift_right_logical(idx_v[pl.ds(o + v * LANES, LANES)], 1)
            pltpu.async_copy(tok_hbm.at[half[b]], rows[b], gsem[b])

        def drain_gather(b):
            pltpu.make_async_copy(tok_hbm.at[pl.ds(0, CHUNK)], rows[b], gsem[b]).wait()

        def drain_writeback(ob):
            pltpu.make_async_copy(outs[ob], out_hbm.at[pl.ds(0, CHUNK)], wsem[ob]).wait()

        for b in range(NB - 1):
            fire_gather(b, b)

        def group_body(g, carry):
            for b in range(NB):
                k = g * NB + b
                drain_gather(b)
                ob = b % NO

                @pl.when(k >= NO)
                def _():
                    drain_writeback(ob)

                def block_body(j, sv):
                    pv16 = (idx_v[pl.ds(k * CHUNK + j * LANES, LANES)] & 1) * e
                    for jj in range(LANES):
                        r = j * LANES + jj
                        par = pv16[jj]
                        svj = sv + jj
                        svj = jnp.where(svj >= s, svj - s, svj)
                        for v in range(e // LANES):
                            sl = pl.ds(v * LANES, LANES)
                            vals = rows[b][r, pl.ds(par + v * LANES, LANES)]
                            outs[ob][r, sl] = vals + pos_v[svj, sl]
                    sv = sv + LANES
                    return jnp.where(sv >= s, sv - s, sv)

                lax.fori_loop(0, CHUNK // LANES, block_body, lax.rem(k * CHUNK, s))
                pltpu.async_copy(
                    outs[ob], out_hbm.at[pl.ds(base_w + k * CHUNK, CHUNK)], wsem[ob]
                )
                kn = k + NB - 1
                bp = (b + NB - 1) % NB

                @pl.when(kn < n_chunks)
                def _():
                    fire_gather(kn, bp)

            return carry

        lax.fori_loop(0, n_groups, group_body, 0)
        for ob in range(NO):
            drain_writeback(ob)

    return embed(ids_flat, token_table2, pos_table)


def kernel(token_ids, token_table, pos_table):
    b, s = token_ids.shape
    v, e = token_table.shape
    n = b * s
    ids_flat = token_ids.reshape(n).astype(jnp.int32)
    tok2 = token_table.reshape(v // 2, 2 * e)
    out = _embed_call(ids_flat, tok2, pos_table, n, s, e)
    return out.reshape(b, s, e)


# speedup vs baseline: 2.9868x; 1.0863x over previous
import functools

import jax
import jax.numpy as jnp
from jax import lax
from jax.experimental import pallas as pl
from jax.experimental.pallas import tpu as pltpu
from jax.experimental.pallas import tpu_sc as plsc

NC, NS = 2, 16
NW = NC * NS
LANES = 16
CHUNK = 200
NB = 4
G1, G2 = 128, 72


def _embed_call(ids_flat, token_table, pos_table, n, s, e):
    per_w = n // NW
    n_chunks = per_w // CHUNK
    assert n_chunks % NB == 0, (n_chunks, NB)
    n_groups = n_chunks // NB

    mesh = plsc.VectorSubcoreMesh(
        core_axis_name="c", subcore_axis_name="s", num_cores=NC, num_subcores=NS
    )

    @functools.partial(
        pl.kernel,
        out_type=jax.ShapeDtypeStruct((n, e), jnp.float32),
        mesh=mesh,
        scratch_types=[
            pltpu.VMEM((per_w,), jnp.int32),
            pltpu.VMEM((s, e), jnp.float32),
        ]
        + [pltpu.VMEM((CHUNK, e), jnp.float32) for _ in range(NB)]
        + [pltpu.SemaphoreType.DMA for _ in range(3 * NB)],
        compiler_params=pltpu.CompilerParams(use_tc_tiling_on_sc=False),
    )
    def embed(ids_hbm, tok_hbm, pos_hbm, out_hbm, idx_v, pos_v, *bufs):
        rows = bufs[:NB]
        gsem1 = bufs[NB : 2 * NB]
        gsem2 = bufs[2 * NB : 3 * NB]
        wsem = bufs[3 * NB : 4 * NB]
        wid = lax.axis_index("s") * NC + lax.axis_index("c")
        base_w = wid * per_w
        pltpu.sync_copy(pos_hbm, pos_v)
        pltpu.sync_copy(ids_hbm.at[pl.ds(base_w, per_w)], idx_v)

        def fire_gather(k, b):
            o = k * CHUNK
            pltpu.async_copy(
                tok_hbm.at[idx_v.at[pl.ds(o, G1)]], rows[b].at[pl.ds(0, G1)], gsem1[b]
            )
            pltpu.async_copy(
                tok_hbm.at[idx_v.at[pl.ds(o + G1, G2)]],
                rows[b].at[pl.ds(G1, G2)],
                gsem2[b],
            )

        def drain_gather(b):
            pltpu.make_async_copy(
                tok_hbm.at[pl.ds(0, G1)], rows[b].at[pl.ds(0, G1)], gsem1[b]
            ).wait()
            pltpu.make_async_copy(
                tok_hbm.at[pl.ds(0, G2)], rows[b].at[pl.ds(G1, G2)], gsem2[b]
            ).wait()

        def drain_writeback(b):
            pltpu.make_async_copy(rows[b], out_hbm.at[pl.ds(0, CHUNK)], wsem[b]).wait()

        for b in range(NB - 1):
            fire_gather(b, b)

        def group_body(g, carry):
            for b in range(NB):
                k = g * NB + b
                drain_gather(b)

                def s_body(si, carry2):
                    for v in range(e // LANES):
                        sl = pl.ds(v * LANES, LANES)
                        plsc.addupdate(rows[b].at[si, sl], pos_v[si, sl])
                    return carry2

                lax.fori_loop(0, CHUNK, s_body, 0)
                pltpu.async_copy(
                    rows[b], out_hbm.at[pl.ds(base_w + k * CHUNK, CHUNK)], wsem[b]
                )
                kn = k + NB - 1
                bp = (b + NB - 1) % NB

                @pl.when(jnp.logical_and(k >= 1, kn < n_chunks))
                def _():
                    drain_writeback(bp)

                @pl.when(kn < n_chunks)
                def _():
                    fire_gather(kn, bp)

            return carry

        lax.fori_loop(0, n_groups, group_body, 0)
        for b in range(NB):
            drain_writeback(b)

    return embed(ids_flat, token_table, pos_table)


def kernel(token_ids, token_table, pos_table):
    b, s = token_ids.shape
    v, e = token_table.shape
    n = b * s
    ids_flat = token_ids.reshape(n).astype(jnp.int32)
    out = _embed_call(ids_flat, token_table, pos_table, n, s, e)
    return out.reshape(b, s, e)
